# baseline (speedup 1.0000x reference)
import functools

import jax
import jax.numpy as jnp
from jax import lax
from jax.experimental import pallas as pl
from jax.experimental.pallas import tpu as pltpu
from jax.experimental.pallas import tpu_sc as plsc

VOCAB = 100000
DIM = 64
EMBED_P = 0.1
BATCH = 4096
HIST = 200

_B = BATCH * HIST

_info = plsc.get_sparse_core_info()
_NC = _info.num_cores
_NS = _info.num_subcores
_NW = _NC * _NS
_BPW = _B // _NW
_CH = 128
_NCHUNK = _BPW // _CH
_NBUF = 4
_LOOK = 2
_NBT = BATCH // _CH

_MROWS = 4000


def _scale_body(w_ref, m_ref, o_ref):
    m_row = m_ref[...].reshape(1, _MROWS)
    m_col = lax.transpose(m_row, (1, 0))
    o_ref[...] = w_ref[...] * m_col


def _masked_table(weight, mask_lanes):
    grid = VOCAB // _MROWS
    return pl.pallas_call(
        _scale_body,
        grid=(grid,),
        in_specs=[
            pl.BlockSpec((_MROWS, DIM), lambda i: (i, 0)),
            pl.BlockSpec((1, 1, _MROWS), lambda i: (i, 0, 0)),
        ],
        out_specs=pl.BlockSpec((_MROWS, DIM), lambda i: (i, 0)),
        out_shape=jax.ShapeDtypeStruct((VOCAB, DIM), jnp.float32),
    )(weight, mask_lanes)


_mesh = plsc.VectorSubcoreMesh(core_axis_name="c", subcore_axis_name="s")


@functools.partial(
    pl.kernel,
    mesh=_mesh,
    out_type=jax.ShapeDtypeStruct((HIST, DIM // 8, _NBT, 8, _CH), jnp.float32),
    scratch_types=[
        pltpu.VMEM((_NCHUNK, _CH), jnp.int32),
    ]
    + [pltpu.VMEM((_CH, DIM), jnp.float32) for _ in range(_NBUF)]
    + [pltpu.VMEM((DIM // 8, 8, _CH), jnp.float32) for _ in range(_NBUF)]
    + [pltpu.SemaphoreType.DMA for _ in range(2 * _NBUF)],
    compiler_params=pltpu.CompilerParams(
        use_tc_tiling_on_sc=False, needs_layout_passes=False
    ),
)
def _sc_gather(tab_hbm, idx_hbm, out_hbm, idx_v, *bufs_and_sems):
    rows = bufs_and_sems[:_NBUF]
    tbuf = bufs_and_sems[_NBUF:2 * _NBUF]
    gsem = bufs_and_sems[2 * _NBUF:3 * _NBUF]
    ssem = bufs_and_sems[3 * _NBUF:]
    wid = lax.axis_index("s") * _NC + lax.axis_index("c")
    base_c = wid * _NCHUNK
    pltpu.sync_copy(idx_hbm.at[wid], idx_v)

    lane = lax.iota(jnp.int32, 16)

    def start_gather(j, b):
        pltpu.async_copy(tab_hbm.at[idx_v.at[j]], rows[b], gsem[b])

    def wait_gather(j, b):
        pltpu.make_async_copy(tab_hbm.at[idx_v.at[j]], rows[b], gsem[b]).wait()

    def transpose_chunk(b):
        def dt_body(dt, _):
            for ds in range(8):
                col = dt * 8 + ds
                for g in range(8):
                    i0 = g * 16 + lane
                    i1 = jnp.full((16,), col, jnp.int32)
                    vals = plsc.load_gather(rows[b], [i0, i1])
                    tbuf[b][dt, ds, pl.ds(g * 16, 16)] = vals
            return 0

        lax.fori_loop(0, DIM // 8, dt_body, 0)

    def _dst(j):
        c = base_c + j
        return c // _NBT, c % _NBT

    def start_scatter(j, b):
        h, bt = _dst(j)
        pltpu.async_copy(
            tbuf[b], out_hbm.at[h, pl.ds(0, DIM // 8), bt], ssem[b]
        )

    def wait_scatter(j, b):
        h, bt = _dst(j)
        pltpu.make_async_copy(
            tbuf[b], out_hbm.at[h, pl.ds(0, DIM // 8), bt], ssem[b]
        ).wait()

    def step(j, b):
        wait_gather(j, b)
        transpose_chunk(b)
        start_scatter(j, b)
        pl.when(j >= _LOOK)(
            lambda: wait_scatter(j - _LOOK, (b - _LOOK) % _NBUF)
        )
        pl.when(j + _LOOK < _NCHUNK)(
            lambda: start_gather(j + _LOOK, (b + _LOOK) % _NBUF)
        )

    for b in range(_LOOK):
        start_gather(b, b)

    def round_body(r, _):
        for b in range(_NBUF):
            step(r * _NBUF + b, b)
        return 0

    lax.fori_loop(0, _NCHUNK // _NBUF, round_body, 0)

    for b in range(_NBUF - _LOOK, _NBUF):
        j = _NCHUNK - _NBUF + b
        wait_scatter(j, b)


def kernel(words, weight):
    keep = jax.random.bernoulli(
        jax.random.key(42), 1.0 - EMBED_P, (VOCAB,)
    ).astype(weight.dtype)
    mask_lanes = (keep / (1.0 - EMBED_P)).reshape(VOCAB // _MROWS, 1, _MROWS)
    masked = _masked_table(weight, mask_lanes)
    idx = words.astype(jnp.int32).T.reshape(_NW, _NCHUNK, _CH)
    p5 = _sc_gather(masked, idx)
    return jnp.transpose(p5, (2, 4, 0, 1, 3)).reshape(BATCH, HIST, DIM)


# speedup vs baseline: 1.8829x; 1.8829x over previous
import functools

import jax
import jax.numpy as jnp
from jax import lax
from jax.experimental import pallas as pl
from jax.experimental.pallas import tpu as pltpu
from jax.experimental.pallas import tpu_sc as plsc

VOCAB = 100000
DIM = 64
EMBED_P = 0.1
BATCH = 4096
HIST = 200

_B = BATCH * HIST

_info = plsc.get_sparse_core_info()
_NC = _info.num_cores
_NS = _info.num_subcores
_NW = _NC * _NS
_BPW = _B // _NW
_CH = 128
_NCHUNK = _BPW // _CH
_NBUF = 4
_LOOK = 2
_NBT = BATCH // _CH

_MROWS = 4000


def _scale_body(w_ref, m_ref, o_ref):
    m_row = m_ref[...].reshape(1, _MROWS)
    m_col = lax.transpose(m_row, (1, 0))
    o_ref[...] = w_ref[...] * m_col


def _masked_table(weight, mask_lanes):
    grid = VOCAB // _MROWS
    return pl.pallas_call(
        _scale_body,
        grid=(grid,),
        in_specs=[
            pl.BlockSpec((_MROWS, DIM), lambda i: (i, 0)),
            pl.BlockSpec((1, 1, _MROWS), lambda i: (i, 0, 0)),
        ],
        out_specs=pl.BlockSpec((_MROWS, DIM), lambda i: (i, 0)),
        out_shape=jax.ShapeDtypeStruct((VOCAB, DIM), jnp.float32),
    )(weight, mask_lanes)


_mesh = plsc.VectorSubcoreMesh(core_axis_name="c", subcore_axis_name="s")


@functools.partial(
    pl.kernel,
    mesh=_mesh,
    out_type=jax.ShapeDtypeStruct((HIST, DIM // 8, _NBT, 8, _CH), jnp.float32),
    scratch_types=[
        pltpu.VMEM((_NCHUNK, _CH), jnp.int32),
    ]
    + [pltpu.VMEM((_CH, DIM), jnp.float32) for _ in range(_NBUF)]
    + [pltpu.VMEM((DIM // 8, 8, _CH), jnp.float32) for _ in range(_NBUF)]
    + [pltpu.SemaphoreType.DMA for _ in range(2 * _NBUF)],
    compiler_params=pltpu.CompilerParams(
        use_tc_tiling_on_sc=False, needs_layout_passes=False
    ),
)
def _sc_gather(tab_hbm, idx_hbm, out_hbm, idx_v, *bufs_and_sems):
    rows = bufs_and_sems[:_NBUF]
    tbuf = bufs_and_sems[_NBUF:2 * _NBUF]
    gsem = bufs_and_sems[2 * _NBUF:3 * _NBUF]
    ssem = bufs_and_sems[3 * _NBUF:]
    wid = lax.axis_index("s") * _NC + lax.axis_index("c")
    base_c = wid * _NCHUNK
    pltpu.sync_copy(idx_hbm.at[wid], idx_v)

    lane = lax.iota(jnp.int32, 16)

    def start_gather(j, b):
        pltpu.async_copy(tab_hbm.at[idx_v.at[j]], rows[b], gsem[b])

    def wait_gather(j, b):
        pltpu.make_async_copy(tab_hbm.at[idx_v.at[j]], rows[b], gsem[b]).wait()

    def transpose_chunk(b):
        def r_body(rb, _):
            i_bl = rb * 16 + lane
            for c0 in range(0, DIM, 16):
                for k in range(16):
                    c_v = c0 + ((lane + k) & 15)
                    vals = plsc.load_gather(rows[b], [i_bl, c_v])
                    plsc.store_scatter(
                        tbuf[b], [c_v >> 3, c_v & 7, i_bl], vals
                    )
            return 0

        lax.fori_loop(0, _CH // 16, r_body, 0)

    def _dst(j):
        c = base_c + j
        return c // _NBT, c % _NBT

    def start_scatter(j, b):
        h, bt = _dst(j)
        pltpu.async_copy(
            tbuf[b], out_hbm.at[h, pl.ds(0, DIM // 8), bt], ssem[b]
        )

    def wait_scatter(j, b):
        h, bt = _dst(j)
        pltpu.make_async_copy(
            tbuf[b], out_hbm.at[h, pl.ds(0, DIM // 8), bt], ssem[b]
        ).wait()

    def step(j, b):
        wait_gather(j, b)
        transpose_chunk(b)
        start_scatter(j, b)
        pl.when(j >= _LOOK)(
            lambda: wait_scatter(j - _LOOK, (b - _LOOK) % _NBUF)
        )
        pl.when(j + _LOOK < _NCHUNK)(
            lambda: start_gather(j + _LOOK, (b + _LOOK) % _NBUF)
        )

    for b in range(_LOOK):
        start_gather(b, b)

    def round_body(r, _):
        for b in range(_NBUF):
            step(r * _NBUF + b, b)
        return 0

    lax.fori_loop(0, _NCHUNK // _NBUF, round_body, 0)

    for b in range(_NBUF - _LOOK, _NBUF):
        j = _NCHUNK - _NBUF + b
        wait_scatter(j, b)


def kernel(words, weight):
    keep = jax.random.bernoulli(
        jax.random.key(42), 1.0 - EMBED_P, (VOCAB,)
    ).astype(weight.dtype)
    mask_lanes = (keep / (1.0 - EMBED_P)).reshape(VOCAB // _MROWS, 1, _MROWS)
    masked = _masked_table(weight, mask_lanes)
    idx = words.astype(jnp.int32).T.reshape(_NW, _NCHUNK, _CH)
    p5 = _sc_gather(masked, idx)
    return jnp.transpose(p5, (2, 4, 0, 1, 3)).reshape(BATCH, HIST, DIM)


# speedup vs baseline: 3.7737x; 2.0042x over previous
import functools

import jax
import jax.numpy as jnp
from jax import lax
from jax.experimental import pallas as pl
from jax.experimental.pallas import tpu as pltpu
from jax.experimental.pallas import tpu_sc as plsc

VOCAB = 100000
DIM = 64
EMBED_P = 0.1
BATCH = 4096
HIST = 200

_B = BATCH * HIST

_info = plsc.get_sparse_core_info()
_NC = _info.num_cores
_NS = _info.num_subcores
_NW = _NC * _NS
_BPW = _B // _NW
_CH = 128
_NCHUNK = _BPW // _CH
_NBUF = 4
_LOOK = 2
_NBT = BATCH // _CH

_MROWS = 4000


def _scale_body(w_ref, m_ref, o_ref):
    m_row = m_ref[...].reshape(1, _MROWS)
    m_col = lax.transpose(m_row, (1, 0))
    o_ref[...] = w_ref[...] * m_col


def _masked_table(weight, mask_lanes):
    grid = VOCAB // _MROWS
    return pl.pallas_call(
        _scale_body,
        grid=(grid,),
        in_specs=[
            pl.BlockSpec((_MROWS, DIM), lambda i: (i, 0)),
            pl.BlockSpec((1, 1, _MROWS), lambda i: (i, 0, 0)),
        ],
        out_specs=pl.BlockSpec((_MROWS, DIM), lambda i: (i, 0)),
        out_shape=jax.ShapeDtypeStruct((VOCAB, DIM), jnp.float32),
    )(weight, mask_lanes)


_mesh = plsc.VectorSubcoreMesh(core_axis_name="c", subcore_axis_name="s")


@functools.partial(
    pl.kernel,
    mesh=_mesh,
    out_type=jax.ShapeDtypeStruct((HIST, DIM // 8, _NBT, 8, _CH), jnp.float32),
    scratch_types=[
        pltpu.VMEM((_NCHUNK, _CH), jnp.int32),
    ]
    + [pltpu.VMEM((_CH, DIM), jnp.float32) for _ in range(_NBUF)]
    + [pltpu.VMEM((DIM // 8, 8, _CH), jnp.float32) for _ in range(_NBUF)]
    + [pltpu.SemaphoreType.DMA for _ in range(2 * _NBUF)],
    compiler_params=pltpu.CompilerParams(
        use_tc_tiling_on_sc=False, needs_layout_passes=False
    ),
)
def _sc_gather(tab_hbm, idx_hbm, out_hbm, idx_v, *bufs_and_sems):
    rows = bufs_and_sems[:_NBUF]
    tbuf = bufs_and_sems[_NBUF:2 * _NBUF]
    gsem = bufs_and_sems[2 * _NBUF:3 * _NBUF]
    ssem = bufs_and_sems[3 * _NBUF:]
    wid = lax.axis_index("s") * _NC + lax.axis_index("c")
    base_c = wid * _NCHUNK
    pltpu.sync_copy(idx_hbm.at[wid], idx_v)

    lane = lax.iota(jnp.int32, 16)

    def start_gather(j, b):
        pltpu.async_copy(tab_hbm.at[idx_v.at[j]], rows[b], gsem[b])

    def wait_gather(j, b):
        pltpu.make_async_copy(tab_hbm.at[idx_v.at[j]], rows[b], gsem[b]).wait()

    def transpose_chunk(b):
        def r_body(rb, _):
            i_bl = rb * 16 + lane
            for c0 in range(0, DIM, 16):
                cols = [c0 + ((lane + k) & 15) for k in range(16)]
                vals = [
                    plsc.load_gather(rows[b], [i_bl, c_v]) for c_v in cols
                ]
                for c_v, v in zip(cols, vals):
                    plsc.store_scatter(tbuf[b], [c_v >> 3, c_v & 7, i_bl], v)
            return 0

        lax.fori_loop(0, _CH // 16, r_body, 0)

    def _dst(j):
        c = base_c + j
        return c // _NBT, c % _NBT

    def start_scatter(j, b):
        h, bt = _dst(j)
        pltpu.async_copy(
            tbuf[b], out_hbm.at[h, pl.ds(0, DIM // 8), bt], ssem[b]
        )

    def wait_scatter(j, b):
        h, bt = _dst(j)
        pltpu.make_async_copy(
            tbuf[b], out_hbm.at[h, pl.ds(0, DIM // 8), bt], ssem[b]
        ).wait()

    def step(j, b):
        wait_gather(j, b)
        transpose_chunk(b)
        start_scatter(j, b)
        pl.when(j >= _LOOK)(
            lambda: wait_scatter(j - _LOOK, (b - _LOOK) % _NBUF)
        )
        pl.when(j + _LOOK < _NCHUNK)(
            lambda: start_gather(j + _LOOK, (b + _LOOK) % _NBUF)
        )

    for b in range(_LOOK):
        start_gather(b, b)

    def round_body(r, _):
        for b in range(_NBUF):
            step(r * _NBUF + b, b)
        return 0

    lax.fori_loop(0, _NCHUNK // _NBUF, round_body, 0)

    for b in range(_NBUF - _LOOK, _NBUF):
        j = _NCHUNK - _NBUF + b
        wait_scatter(j, b)


def kernel(words, weight):
    keep = jax.random.bernoulli(
        jax.random.key(42), 1.0 - EMBED_P, (VOCAB,)
    ).astype(weight.dtype)
    mask_lanes = (keep / (1.0 - EMBED_P)).reshape(VOCAB // _MROWS, 1, _MROWS)
    masked = _masked_table(weight, mask_lanes)
    idx = words.astype(jnp.int32).T.reshape(_NW, _NCHUNK, _CH)
    p5 = _sc_gather(masked, idx)
    return jnp.transpose(p5, (2, 4, 0, 1, 3)).reshape(BATCH, HIST, DIM)


# speedup vs baseline: 4.6218x; 1.2247x over previous
import functools

import jax
import jax.numpy as jnp
from jax import lax
from jax.experimental import pallas as pl
from jax.experimental.pallas import tpu as pltpu
from jax.experimental.pallas import tpu_sc as plsc

VOCAB = 100000
DIM = 64
EMBED_P = 0.1
BATCH = 4096
HIST = 200

_B = BATCH * HIST

_info = plsc.get_sparse_core_info()
_NC = _info.num_cores
_NS = _info.num_subcores
_NW = _NC * _NS
_BPW = _B // _NW
_CH = 128
_NCHUNK = _BPW // _CH
_NBUF = 4
_LOOK = 2
_NBT = BATCH // _CH

_MROWS = 4000


def _mask_body(k_ref, o_ref):
    o_ref[...] = k_ref[...] / (1.0 - EMBED_P)


def _mask_values(keep1d):
    return pl.pallas_call(
        _mask_body,
        out_shape=jax.ShapeDtypeStruct((VOCAB,), jnp.float32),
    )(keep1d)


_mesh = plsc.VectorSubcoreMesh(core_axis_name="c", subcore_axis_name="s")


@functools.partial(
    pl.kernel,
    mesh=_mesh,
    out_type=jax.ShapeDtypeStruct((HIST, DIM // 8, _NBT, 8, _CH), jnp.float32),
    scratch_types=[
        pltpu.VMEM((_NCHUNK, _CH), jnp.int32),
    ]
    + [pltpu.VMEM((_CH, DIM), jnp.float32) for _ in range(_NBUF)]
    + [pltpu.VMEM((DIM // 8, 8, _CH), jnp.float32) for _ in range(_NBUF)]
    + [pltpu.VMEM((_CH,), jnp.float32) for _ in range(_NBUF)]
    + [pltpu.SemaphoreType.DMA for _ in range(3 * _NBUF)],
    compiler_params=pltpu.CompilerParams(
        use_tc_tiling_on_sc=False, needs_layout_passes=False
    ),
)
def _sc_gather(tab_hbm, idx_hbm, msk_hbm, out_hbm, idx_v, *bufs_and_sems):
    rows = bufs_and_sems[:_NBUF]
    tbuf = bufs_and_sems[_NBUF:2 * _NBUF]
    mval = bufs_and_sems[2 * _NBUF:3 * _NBUF]
    gsem = bufs_and_sems[3 * _NBUF:4 * _NBUF]
    ssem = bufs_and_sems[4 * _NBUF:5 * _NBUF]
    msem = bufs_and_sems[5 * _NBUF:]
    wid = lax.axis_index("s") * _NC + lax.axis_index("c")
    base_c = wid * _NCHUNK
    pltpu.sync_copy(idx_hbm.at[wid], idx_v)

    lane = lax.iota(jnp.int32, 16)

    def start_gather(j, b):
        pltpu.async_copy(tab_hbm.at[idx_v.at[j]], rows[b], gsem[b])
        pltpu.async_copy(msk_hbm.at[idx_v.at[j]], mval[b], msem[b])

    def wait_gather(j, b):
        pltpu.make_async_copy(tab_hbm.at[idx_v.at[j]], rows[b], gsem[b]).wait()
        pltpu.make_async_copy(
            msk_hbm.at[idx_v.at[j]], mval[b], msem[b]
        ).wait()

    def transpose_chunk(b):
        def r_body(rb, _):
            i_bl = rb * 16 + lane
            mv = plsc.load_gather(mval[b], [i_bl])
            for c0 in range(0, DIM, 16):
                cols = [c0 + ((lane + k) & 15) for k in range(16)]
                vals = [
                    plsc.load_gather(rows[b], [i_bl, c_v]) * mv
                    for c_v in cols
                ]
                for c_v, v in zip(cols, vals):
                    plsc.store_scatter(tbuf[b], [c_v >> 3, c_v & 7, i_bl], v)
            return 0

        lax.fori_loop(0, _CH // 16, r_body, 0)

    def _dst(j):
        c = base_c + j
        return c // _NBT, c % _NBT

    def start_scatter(j, b):
        h, bt = _dst(j)
        pltpu.async_copy(
            tbuf[b], out_hbm.at[h, pl.ds(0, DIM // 8), bt], ssem[b]
        )

    def wait_scatter(j, b):
        h, bt = _dst(j)
        pltpu.make_async_copy(
            tbuf[b], out_hbm.at[h, pl.ds(0, DIM // 8), bt], ssem[b]
        ).wait()

    def step(j, b):
        wait_gather(j, b)
        transpose_chunk(b)
        start_scatter(j, b)
        pl.when(j >= _LOOK)(
            lambda: wait_scatter(j - _LOOK, (b - _LOOK) % _NBUF)
        )
        pl.when(j + _LOOK < _NCHUNK)(
            lambda: start_gather(j + _LOOK, (b + _LOOK) % _NBUF)
        )

    for b in range(_LOOK):
        start_gather(b, b)

    def round_body(r, _):
        for b in range(_NBUF):
            step(r * _NBUF + b, b)
        return 0

    lax.fori_loop(0, _NCHUNK // _NBUF, round_body, 0)

    for b in range(_NBUF - _LOOK, _NBUF):
        j = _NCHUNK - _NBUF + b
        wait_scatter(j, b)


def kernel(words, weight):
    keep1d = jax.random.bernoulli(
        jax.random.key(42), 1.0 - EMBED_P, (VOCAB,)
    ).astype(weight.dtype)
    mvals = _mask_values(keep1d)
    idx = words.astype(jnp.int32).T.reshape(_NW, _NCHUNK, _CH)
    p5 = _sc_gather(weight, idx, mvals)
    return jnp.transpose(p5, (2, 4, 0, 1, 3)).reshape(BATCH, HIST, DIM)
